# baseline (speedup 1.0000x reference)
import functools

import jax
import jax.numpy as jnp
from jax import lax
from jax.experimental import pallas as pl
from jax.experimental.pallas import tpu as pltpu
from jax.experimental.pallas import tpu_sc as plsc

N_NODES = 10000
NPAD = 10240
E = 320000
NC, NS = 2, 16
NW = NC * NS
C, B = 80, 128
EPAD = NW * C * B
RPT = NPAD // NS
ZR = 128
DH = 64
D2 = 48
DD = 16

_MESH = plsc.VectorSubcoreMesh(core_axis_name="core", subcore_axis_name="subcore")


def _make_propagate(D, n_phases):
    out_t = [jax.ShapeDtypeStruct((NC, NPAD, D), jnp.float32)] * n_phases

    @functools.partial(
        pl.kernel,
        out_type=out_t if n_phases > 1 else out_t[0],
        mesh=_MESH,
        compiler_params=pltpu.CompilerParams(use_tc_tiling_on_sc=False),
        scratch_types=[
            pltpu.VMEM((C, B), jnp.int32),
            pltpu.VMEM((C, B), jnp.int32),
            pltpu.VMEM((2, B, D), jnp.float32),
            pltpu.VMEM((ZR, D), jnp.float32),
            pltpu.VMEM_SHARED((NPAD, D), jnp.float32),
            pltpu.SemaphoreType.DMA,
            pltpu.SemaphoreType.DMA,
        ],
    )
    def prop(*refs):
        g_hbm = refs[:n_phases]
        src_hbm, dst_hbm = refs[n_phases], refs[n_phases + 1]
        out_hbm = refs[n_phases + 2:2 * n_phases + 2]
        src_v, dst_v, rows_v, zbuf, acc, sem0, sem1 = refs[2 * n_phases + 2:]

        cid = lax.axis_index("core")
        sid = lax.axis_index("subcore")
        wid = sid * NC + cid

        @pl.loop(0, ZR)
        def _(r):
            @pl.loop(0, D, step=16)
            def _(k):
                zbuf[r, pl.ds(k, 16)] = jnp.zeros((16,), jnp.float32)

        pltpu.sync_copy(src_hbm.at[wid], src_v)
        pltpu.sync_copy(dst_hbm.at[wid], dst_v)

        for phase in range(n_phases):
            g_p = g_hbm[phase]
            out_p = out_hbm[phase]

            @pl.loop(0, RPT, step=ZR)
            def _(r0):
                pltpu.sync_copy(zbuf, acc.at[pl.ds(sid * RPT + r0, ZR)])

            plsc.subcore_barrier()

            pltpu.async_copy(g_p.at[src_v.at[0]], rows_v.at[0], sem0)
            pltpu.async_copy(g_p.at[src_v.at[1]], rows_v.at[1], sem1)

            @pl.loop(0, C, step=2)
            def _(j):
                for slot, sem in ((0, sem0), (1, sem1)):
                    jj = j + slot

                    @pl.when(jj < C)
                    def _():
                        pltpu.make_async_copy(
                            g_p.at[src_v.at[jj]], rows_v.at[slot], sem).wait()
                        pltpu.sync_copy(
                            rows_v.at[slot], acc.at[dst_v.at[jj]], add=True)

                        @pl.when(jj + 2 < C)
                        def _():
                            pltpu.async_copy(
                                g_p.at[src_v.at[jj + 2]], rows_v.at[slot], sem)

            plsc.subcore_barrier()
            pltpu.sync_copy(
                acc.at[pl.ds(sid * RPT, RPT)],
                out_p.at[cid, pl.ds(sid * RPT, RPT)],
            )

    return prop


_prop_l1 = _make_propagate(DH, 2)
_prop_l2 = _make_propagate(D2, 1)


@functools.partial(
    pl.kernel,
    out_type=jax.ShapeDtypeStruct((NC, NPAD, DD), jnp.float32),
    mesh=_MESH,
    compiler_params=pltpu.CompilerParams(use_tc_tiling_on_sc=False),
    scratch_types=[
        pltpu.VMEM((C, B), jnp.int32),
        pltpu.VMEM((B, DD), jnp.float32),
        pltpu.VMEM((ZR, DD), jnp.float32),
        pltpu.VMEM_SHARED((NPAD, DD), jnp.float32),
    ],
)
def _degree(dst_hbm, out_hbm, dst_v, ones_v, zbuf, acc):
    cid = lax.axis_index("core")
    sid = lax.axis_index("subcore")
    wid = sid * NC + cid

    @pl.loop(0, ZR)
    def _(r):
        zbuf[r, pl.ds(0, 16)] = jnp.zeros((16,), jnp.float32)

    @pl.loop(0, B)
    def _(r):
        ones_v[r, pl.ds(0, 16)] = jnp.full((16,), 1.0 / 16.0, jnp.float32)

    @pl.loop(0, RPT, step=ZR)
    def _(r0):
        pltpu.sync_copy(zbuf, acc.at[pl.ds(sid * RPT + r0, ZR)])

    pltpu.sync_copy(dst_hbm.at[wid], dst_v)
    plsc.subcore_barrier()

    @pl.loop(0, C)
    def _(j):
        pltpu.sync_copy(ones_v, acc.at[dst_v.at[j]], add=True)

    plsc.subcore_barrier()
    pltpu.sync_copy(
        acc.at[pl.ds(sid * RPT, RPT)],
        out_hbm.at[cid, pl.ds(sid * RPT, RPT)],
    )


RB = 2048


def _dis_from_parts(d):
    deg = jnp.sum(d[0], axis=1) + jnp.sum(d[1], axis=1) + 1.0
    return lax.rsqrt(deg)


def _tc_first(x_ref, w_ref, d_ref, oa_ref, ob_ref):
    dis = _dis_from_parts(d_ref[...])
    y = jnp.dot(x_ref[...], w_ref[...], preferred_element_type=jnp.float32,
                precision=lax.Precision.HIGHEST)
    g = y * dis[:, None]
    oa_ref[...] = g[:, :DH]
    ob_ref[...] = g[:, DH:]


def _tc_mid(pa_ref, pb_ref, ga_ref, gb_ref, d_ref, b_ref, w_ref, o_ref):
    dis = _dis_from_parts(d_ref[...])[:, None]
    sa = pa_ref[0] + pa_ref[1] + ga_ref[...]
    sb = pb_ref[0] + pb_ref[1] + gb_ref[...]
    s = jnp.concatenate([sa, sb], axis=1)
    h = jnp.maximum(dis * s + b_ref[...], 0.0)
    y = jnp.dot(h, w_ref[...], preferred_element_type=jnp.float32,
                precision=lax.Precision.HIGHEST)
    o_ref[...] = y * dis


def _tc_last(p_ref, g_ref, d_ref, b_ref, o_ref):
    dis = _dis_from_parts(d_ref[...])[:, None]
    s = p_ref[0] + p_ref[1] + g_ref[...]
    o_ref[...] = dis * s + b_ref[...]


def _row_spec(d):
    return pl.BlockSpec((RB, d), lambda i: (i, 0))


def _part_spec(d):
    return pl.BlockSpec((NC, RB, d), lambda i: (0, i, 0))


def _full_spec(shape):
    return pl.BlockSpec(shape, lambda i: tuple(0 for _ in shape))


@jax.jit
def kernel(x, edge_index, W1, b1, W2, b2):
    ei = edge_index.astype(jnp.int32)
    pad_src = jnp.full((EPAD - E,), N_NODES, jnp.int32)
    pad_dst = jnp.full((EPAD - E,), NPAD - 1, jnp.int32)
    src = jnp.concatenate([ei[0], pad_src]).reshape(NW, C, B)
    dst = jnp.concatenate([ei[1], pad_dst]).reshape(NW, C, B)
    xp = jnp.pad(x, ((0, NPAD - N_NODES), (0, 0)))
    w2p = jnp.pad(W2, ((0, 0), (0, D2 - W2.shape[1])))
    b1r = b1.reshape(1, 2 * DH)
    b2r = jnp.pad(b2, (0, D2 - b2.shape[0])).reshape(1, D2)

    degp = _degree(dst)

    nblk = NPAD // RB

    g1a, g1b = pl.pallas_call(
        _tc_first,
        grid=(nblk,),
        in_specs=[_row_spec(2 * DH), _full_spec((2 * DH, 2 * DH)), _part_spec(DD)],
        out_specs=[_row_spec(DH), _row_spec(DH)],
        out_shape=[jax.ShapeDtypeStruct((NPAD, DH), jnp.float32)] * 2,
    )(xp, W1, degp)

    p1a, p1b = _prop_l1(g1a, g1b, src, dst)

    g2 = pl.pallas_call(
        _tc_mid,
        grid=(nblk,),
        in_specs=[_part_spec(DH), _part_spec(DH), _row_spec(DH), _row_spec(DH),
                  _part_spec(DD), _full_spec((1, 2 * DH)), _full_spec((2 * DH, D2))],
        out_specs=_row_spec(D2),
        out_shape=jax.ShapeDtypeStruct((NPAD, D2), jnp.float32),
    )(p1a, p1b, g1a, g1b, degp, b1r, w2p)

    p2 = _prop_l2(g2, src, dst)

    outp = pl.pallas_call(
        _tc_last,
        grid=(nblk,),
        in_specs=[_part_spec(D2), _row_spec(D2), _part_spec(DD),
                  _full_spec((1, D2))],
        out_specs=_row_spec(D2),
        out_shape=jax.ShapeDtypeStruct((NPAD, D2), jnp.float32),
    )(p2, g2, degp, b2r)

    return outp[:N_NODES, :40]


# speedup vs baseline: 2.2704x; 2.2704x over previous
import functools

import jax
import jax.numpy as jnp
from jax import lax
from jax.experimental import pallas as pl
from jax.experimental.pallas import tpu as pltpu
from jax.experimental.pallas import tpu_sc as plsc

N_NODES = 10000
NPAD = 10240
E = 320000
NC, NS = 2, 16
NW = NC * NS
C, B = 80, 128
EPAD = NW * C * B
RPT = NPAD // NS
ZR = 128
DH = 64
D2 = 48
DD = 16

_MESH = plsc.VectorSubcoreMesh(core_axis_name="core", subcore_axis_name="subcore")


def _make_propagate(D, n_phases):
    out_t = [jax.ShapeDtypeStruct((NC, NPAD, D), jnp.float32)] * n_phases

    @functools.partial(
        pl.kernel,
        out_type=out_t if n_phases > 1 else out_t[0],
        mesh=_MESH,
        compiler_params=pltpu.CompilerParams(use_tc_tiling_on_sc=False),
        scratch_types=[
            pltpu.VMEM((C, B), jnp.int32),
            pltpu.VMEM((C, B), jnp.int32),
            pltpu.VMEM((2, B, D), jnp.float32),
            pltpu.VMEM((ZR, D), jnp.float32),
            pltpu.VMEM_SHARED((NPAD, D), jnp.float32),
            pltpu.SemaphoreType.DMA,
            pltpu.SemaphoreType.DMA,
        ],
    )
    def prop(*refs):
        g_hbm = refs[:n_phases]
        src_hbm, dst_hbm = refs[n_phases], refs[n_phases + 1]
        out_hbm = refs[n_phases + 2:2 * n_phases + 2]
        src_v, dst_v, rows_v, zbuf, acc, sem0, sem1 = refs[2 * n_phases + 2:]

        cid = lax.axis_index("core")
        sid = lax.axis_index("subcore")
        wid = sid * NC + cid

        @pl.loop(0, ZR)
        def _(r):
            @pl.loop(0, D, step=16)
            def _(k):
                zbuf[r, pl.ds(k, 16)] = jnp.zeros((16,), jnp.float32)

        pltpu.sync_copy(src_hbm.at[wid], src_v)
        pltpu.sync_copy(dst_hbm.at[wid], dst_v)

        for phase in range(n_phases):
            g_p = g_hbm[phase]
            out_p = out_hbm[phase]

            @pl.loop(0, RPT, step=ZR)
            def _(r0):
                pltpu.sync_copy(zbuf, acc.at[pl.ds(sid * RPT + r0, ZR)])

            plsc.subcore_barrier()

            pltpu.async_copy(g_p.at[src_v.at[0]], rows_v.at[0], sem0)
            pltpu.async_copy(g_p.at[src_v.at[1]], rows_v.at[1], sem1)

            @pl.loop(0, C, step=2)
            def _(j):
                for slot, sem in ((0, sem0), (1, sem1)):
                    jj = j + slot

                    @pl.when(jj < C)
                    def _():
                        pltpu.make_async_copy(
                            g_p.at[src_v.at[jj]], rows_v.at[slot], sem).wait()
                        pltpu.sync_copy(
                            rows_v.at[slot], acc.at[dst_v.at[jj]], add=True)

                        @pl.when(jj + 2 < C)
                        def _():
                            pltpu.async_copy(
                                g_p.at[src_v.at[jj + 2]], rows_v.at[slot], sem)

            plsc.subcore_barrier()
            pltpu.sync_copy(
                acc.at[pl.ds(sid * RPT, RPT)],
                out_p.at[cid, pl.ds(sid * RPT, RPT)],
            )

    return prop


_prop_l1 = _make_propagate(DH, 2)
_prop_l2 = _make_propagate(D2, 1)


@functools.partial(
    pl.kernel,
    out_type=jax.ShapeDtypeStruct((NC, NPAD, DD), jnp.float32),
    mesh=_MESH,
    compiler_params=pltpu.CompilerParams(use_tc_tiling_on_sc=False),
    scratch_types=[
        pltpu.VMEM((C, B), jnp.int32),
        pltpu.VMEM((B, DD), jnp.float32),
        pltpu.VMEM((ZR, DD), jnp.float32),
        pltpu.VMEM_SHARED((NPAD, DD), jnp.float32),
    ],
)
def _degree(dst_hbm, out_hbm, dst_v, ones_v, zbuf, acc):
    cid = lax.axis_index("core")
    sid = lax.axis_index("subcore")
    wid = sid * NC + cid

    @pl.loop(0, ZR)
    def _(r):
        zbuf[r, pl.ds(0, 16)] = jnp.zeros((16,), jnp.float32)

    @pl.loop(0, B)
    def _(r):
        ones_v[r, pl.ds(0, 16)] = jnp.full((16,), 1.0 / 16.0, jnp.float32)

    @pl.loop(0, RPT, step=ZR)
    def _(r0):
        pltpu.sync_copy(zbuf, acc.at[pl.ds(sid * RPT + r0, ZR)])

    pltpu.sync_copy(dst_hbm.at[wid], dst_v)
    plsc.subcore_barrier()

    @pl.loop(0, C)
    def _(j):
        pltpu.sync_copy(ones_v, acc.at[dst_v.at[j]], add=True)

    plsc.subcore_barrier()
    pltpu.sync_copy(
        acc.at[pl.ds(sid * RPT, RPT)],
        out_hbm.at[cid, pl.ds(sid * RPT, RPT)],
    )


RB = 2048


def _dis_from_parts(d):
    deg = jnp.sum(d[0], axis=1) + jnp.sum(d[1], axis=1) + 1.0
    return lax.rsqrt(deg)


def _tc_first(x_ref, w_ref, d_ref, oa_ref, ob_ref):
    dis = _dis_from_parts(d_ref[...])
    y = jnp.dot(x_ref[...], w_ref[...], preferred_element_type=jnp.float32,
                precision=lax.Precision.HIGHEST)
    g = y * dis[:, None]
    oa_ref[...] = g[:, :DH]
    ob_ref[...] = g[:, DH:]


def _tc_mid(pa_ref, pb_ref, ga_ref, gb_ref, d_ref, b_ref, w_ref, o_ref):
    dis = _dis_from_parts(d_ref[...])[:, None]
    sa = pa_ref[0] + pa_ref[1] + ga_ref[...]
    sb = pb_ref[0] + pb_ref[1] + gb_ref[...]
    s = jnp.concatenate([sa, sb], axis=1)
    h = jnp.maximum(dis * s + b_ref[...], 0.0)
    y = jnp.dot(h, w_ref[...], preferred_element_type=jnp.float32,
                precision=lax.Precision.HIGHEST)
    o_ref[...] = y * dis


def _tc_last(p_ref, g_ref, d_ref, b_ref, o_ref):
    dis = _dis_from_parts(d_ref[...])[:, None]
    s = p_ref[0] + p_ref[1] + g_ref[...]
    o_ref[...] = dis * s + b_ref[...]


def _row_spec(d):
    return pl.BlockSpec((RB, d), lambda i: (i, 0))


def _part_spec(d):
    return pl.BlockSpec((NC, RB, d), lambda i: (0, i, 0))


def _full_spec(shape):
    return pl.BlockSpec(shape, lambda i: tuple(0 for _ in shape))


@jax.jit
def kernel(x, edge_index, W1, b1, W2, b2):
    ei = edge_index.astype(jnp.int32)
    pad_src = N_NODES + jnp.arange(EPAD - E, dtype=jnp.int32) % (NPAD - N_NODES)
    pad_dst = N_NODES + jnp.arange(EPAD - E, dtype=jnp.int32) % (NPAD - N_NODES)
    src = jnp.concatenate([ei[0], pad_src]).reshape(NW, C, B)
    dst = jnp.concatenate([ei[1], pad_dst]).reshape(NW, C, B)
    xp = jnp.pad(x, ((0, NPAD - N_NODES), (0, 0)))
    w2p = jnp.pad(W2, ((0, 0), (0, D2 - W2.shape[1])))
    b1r = b1.reshape(1, 2 * DH)
    b2r = jnp.pad(b2, (0, D2 - b2.shape[0])).reshape(1, D2)

    degp = _degree(dst)

    nblk = NPAD // RB

    g1a, g1b = pl.pallas_call(
        _tc_first,
        grid=(nblk,),
        in_specs=[_row_spec(2 * DH), _full_spec((2 * DH, 2 * DH)), _part_spec(DD)],
        out_specs=[_row_spec(DH), _row_spec(DH)],
        out_shape=[jax.ShapeDtypeStruct((NPAD, DH), jnp.float32)] * 2,
    )(xp, W1, degp)

    p1a, p1b = _prop_l1(g1a, g1b, src, dst)

    g2 = pl.pallas_call(
        _tc_mid,
        grid=(nblk,),
        in_specs=[_part_spec(DH), _part_spec(DH), _row_spec(DH), _row_spec(DH),
                  _part_spec(DD), _full_spec((1, 2 * DH)), _full_spec((2 * DH, D2))],
        out_specs=_row_spec(D2),
        out_shape=jax.ShapeDtypeStruct((NPAD, D2), jnp.float32),
    )(p1a, p1b, g1a, g1b, degp, b1r, w2p)

    p2 = _prop_l2(g2, src, dst)

    outp = pl.pallas_call(
        _tc_last,
        grid=(nblk,),
        in_specs=[_part_spec(D2), _row_spec(D2), _part_spec(DD),
                  _full_spec((1, D2))],
        out_specs=_row_spec(D2),
        out_shape=jax.ShapeDtypeStruct((NPAD, D2), jnp.float32),
    )(p2, g2, degp, b2r)

    return outp[:N_NODES, :40]


# speedup vs baseline: 2.3517x; 1.0358x over previous
import functools

import jax
import jax.numpy as jnp
from jax import lax
from jax.experimental import pallas as pl
from jax.experimental.pallas import tpu as pltpu
from jax.experimental.pallas import tpu_sc as plsc

N_NODES = 10000
NPAD = 10240
E = 320000
NC, NS = 2, 16
NW = NC * NS
C, B = 80, 128
EPAD = NW * C * B
RPT = NPAD // NS
ZR = 128
DH = 64
D2 = 48
DD = 16

_MESH = plsc.VectorSubcoreMesh(core_axis_name="core", subcore_axis_name="subcore")


def _make_propagate(D, n_phases):
    out_t = [jax.ShapeDtypeStruct((NC, NPAD, D), jnp.float32)] * n_phases

    @functools.partial(
        pl.kernel,
        out_type=out_t if n_phases > 1 else out_t[0],
        mesh=_MESH,
        compiler_params=pltpu.CompilerParams(use_tc_tiling_on_sc=False),
        scratch_types=[
            pltpu.VMEM((C, B), jnp.int32),
            pltpu.VMEM((C, B), jnp.int32),
            pltpu.VMEM((2, B, D), jnp.float32),
            pltpu.VMEM((ZR, D), jnp.float32),
            pltpu.VMEM_SHARED((NPAD, D), jnp.float32),
            pltpu.SemaphoreType.DMA,
            pltpu.SemaphoreType.DMA,
        ],
    )
    def prop(*refs):
        g_hbm = refs[:n_phases]
        src_hbm, dst_hbm = refs[n_phases], refs[n_phases + 1]
        out_hbm = refs[n_phases + 2:2 * n_phases + 2]
        src_v, dst_v, rows_v, zbuf, acc, sem0, sem1 = refs[2 * n_phases + 2:]

        cid = lax.axis_index("core")
        sid = lax.axis_index("subcore")
        wid = sid * NC + cid

        @pl.loop(0, ZR)
        def _(r):
            @pl.loop(0, D, step=16)
            def _(k):
                zbuf[r, pl.ds(k, 16)] = jnp.zeros((16,), jnp.float32)

        pltpu.sync_copy(src_hbm.at[wid], src_v)
        pltpu.sync_copy(dst_hbm.at[wid], dst_v)

        for phase in range(n_phases):
            g_p = g_hbm[phase]
            out_p = out_hbm[phase]

            @pl.loop(0, RPT, step=ZR)
            def _(r0):
                pltpu.sync_copy(zbuf, acc.at[pl.ds(sid * RPT + r0, ZR)])

            plsc.subcore_barrier()

            pltpu.async_copy(g_p.at[src_v.at[0]], rows_v.at[0], sem0)
            pltpu.async_copy(g_p.at[src_v.at[1]], rows_v.at[1], sem1)

            @pl.loop(0, C, step=2)
            def _(j):
                for slot, sem in ((0, sem0), (1, sem1)):
                    jj = j + slot

                    @pl.when(jj < C)
                    def _():
                        pltpu.make_async_copy(
                            g_p.at[src_v.at[jj]], rows_v.at[slot], sem).wait()
                        pltpu.sync_copy(
                            rows_v.at[slot], acc.at[dst_v.at[jj]], add=True)

                        @pl.when(jj + 2 < C)
                        def _():
                            pltpu.async_copy(
                                g_p.at[src_v.at[jj + 2]], rows_v.at[slot], sem)

            plsc.subcore_barrier()
            pltpu.sync_copy(
                acc.at[pl.ds(sid * RPT, RPT)],
                out_p.at[cid, pl.ds(sid * RPT, RPT)],
            )

    return prop


_prop_l2 = _make_propagate(D2, 1)

C1 = 158
E1PAD = NS * C1 * B


@functools.partial(
    pl.kernel,
    out_type=jax.ShapeDtypeStruct((NC, NPAD, DH), jnp.float32),
    mesh=_MESH,
    compiler_params=pltpu.CompilerParams(use_tc_tiling_on_sc=False),
    scratch_types=[
        pltpu.VMEM((C1, B), jnp.int32),
        pltpu.VMEM((C1, B), jnp.int32),
        pltpu.VMEM((2, B, DH), jnp.float32),
        pltpu.VMEM((ZR, DH), jnp.float32),
        pltpu.VMEM_SHARED((NPAD, DH), jnp.float32),
        pltpu.SemaphoreType.DMA,
        pltpu.SemaphoreType.DMA,
    ],
)
def _prop_l1(g_hbm, src_hbm, dst_hbm, out_hbm, src_v, dst_v, rows_v, zbuf,
             acc, sem0, sem1):
    cid = lax.axis_index("core")
    sid = lax.axis_index("subcore")

    @pl.loop(0, ZR)
    def _(r):
        @pl.loop(0, DH, step=16)
        def _(k):
            zbuf[r, pl.ds(k, 16)] = jnp.zeros((16,), jnp.float32)

    pltpu.sync_copy(src_hbm.at[sid], src_v)
    pltpu.sync_copy(dst_hbm.at[sid], dst_v)
    g_p = g_hbm.at[cid]

    @pl.loop(0, RPT, step=ZR)
    def _(r0):
        pltpu.sync_copy(zbuf, acc.at[pl.ds(sid * RPT + r0, ZR)])

    plsc.subcore_barrier()

    pltpu.async_copy(g_p.at[src_v.at[0]], rows_v.at[0], sem0)
    pltpu.async_copy(g_p.at[src_v.at[1]], rows_v.at[1], sem1)

    @pl.loop(0, C1, step=2)
    def _(j):
        for slot, sem in ((0, sem0), (1, sem1)):
            jj = j + slot
            pltpu.make_async_copy(
                g_p.at[src_v.at[jj]], rows_v.at[slot], sem).wait()
            pltpu.sync_copy(rows_v.at[slot], acc.at[dst_v.at[jj]], add=True)

            @pl.when(jj + 2 < C1)
            def _():
                pltpu.async_copy(
                    g_p.at[src_v.at[jj + 2]], rows_v.at[slot], sem)

    plsc.subcore_barrier()
    pltpu.sync_copy(
        acc.at[pl.ds(sid * RPT, RPT)],
        out_hbm.at[cid, pl.ds(sid * RPT, RPT)],
    )


@functools.partial(
    pl.kernel,
    out_type=jax.ShapeDtypeStruct((NC, NPAD, DD), jnp.float32),
    mesh=_MESH,
    compiler_params=pltpu.CompilerParams(use_tc_tiling_on_sc=False),
    scratch_types=[
        pltpu.VMEM((C, B), jnp.int32),
        pltpu.VMEM((B, DD), jnp.float32),
        pltpu.VMEM((ZR, DD), jnp.float32),
        pltpu.VMEM_SHARED((NPAD, DD), jnp.float32),
    ],
)
def _degree(dst_hbm, out_hbm, dst_v, ones_v, zbuf, acc):
    cid = lax.axis_index("core")
    sid = lax.axis_index("subcore")
    wid = sid * NC + cid

    @pl.loop(0, ZR)
    def _(r):
        zbuf[r, pl.ds(0, 16)] = jnp.zeros((16,), jnp.float32)

    @pl.loop(0, B)
    def _(r):
        ones_v[r, pl.ds(0, 16)] = jnp.full((16,), 1.0 / 16.0, jnp.float32)

    @pl.loop(0, RPT, step=ZR)
    def _(r0):
        pltpu.sync_copy(zbuf, acc.at[pl.ds(sid * RPT + r0, ZR)])

    pltpu.sync_copy(dst_hbm.at[wid], dst_v)
    plsc.subcore_barrier()

    @pl.loop(0, C)
    def _(j):
        pltpu.sync_copy(ones_v, acc.at[dst_v.at[j]], add=True)

    plsc.subcore_barrier()
    pltpu.sync_copy(
        acc.at[pl.ds(sid * RPT, RPT)],
        out_hbm.at[cid, pl.ds(sid * RPT, RPT)],
    )


RB = 2048


def _dis_from_parts(d):
    deg = jnp.sum(d[0], axis=1) + jnp.sum(d[1], axis=1) + 1.0
    return lax.rsqrt(deg)


def _tc_first(x_ref, w_ref, d_ref, o_ref):
    dis = _dis_from_parts(d_ref[...])
    y = jnp.dot(x_ref[...], w_ref[...], preferred_element_type=jnp.float32,
                precision=lax.Precision.HIGHEST)
    g = y * dis[:, None]
    o_ref[0] = g[:, :DH]
    o_ref[1] = g[:, DH:]


def _tc_mid(p_ref, g_ref, d_ref, b_ref, w_ref, o_ref):
    dis = _dis_from_parts(d_ref[...])[:, None]
    sa = p_ref[0] + g_ref[0]
    sb = p_ref[1] + g_ref[1]
    s = jnp.concatenate([sa, sb], axis=1)
    h = jnp.maximum(dis * s + b_ref[...], 0.0)
    y = jnp.dot(h, w_ref[...], preferred_element_type=jnp.float32,
                precision=lax.Precision.HIGHEST)
    o_ref[...] = y * dis


def _tc_last(p_ref, g_ref, d_ref, b_ref, o_ref):
    dis = _dis_from_parts(d_ref[...])[:, None]
    s = p_ref[0] + p_ref[1] + g_ref[...]
    o_ref[...] = dis * s + b_ref[...]


def _row_spec(d):
    return pl.BlockSpec((RB, d), lambda i: (i, 0))


def _part_spec(d):
    return pl.BlockSpec((NC, RB, d), lambda i: (0, i, 0))


def _full_spec(shape):
    return pl.BlockSpec(shape, lambda i: tuple(0 for _ in shape))


@jax.jit
def kernel(x, edge_index, W1, b1, W2, b2):
    ei = edge_index.astype(jnp.int32)
    pad_src = N_NODES + jnp.arange(EPAD - E, dtype=jnp.int32) % (NPAD - N_NODES)
    pad_dst = N_NODES + jnp.arange(EPAD - E, dtype=jnp.int32) % (NPAD - N_NODES)
    src = jnp.concatenate([ei[0], pad_src]).reshape(NW, C, B)
    dst = jnp.concatenate([ei[1], pad_dst]).reshape(NW, C, B)
    pad1_src = N_NODES + jnp.arange(E1PAD - E, dtype=jnp.int32) % (NPAD - N_NODES)
    pad1_dst = N_NODES + jnp.arange(E1PAD - E, dtype=jnp.int32) % (NPAD - N_NODES)
    src1 = jnp.concatenate([ei[0], pad1_src]).reshape(NS, C1, B)
    dst1 = jnp.concatenate([ei[1], pad1_dst]).reshape(NS, C1, B)
    xp = jnp.pad(x, ((0, NPAD - N_NODES), (0, 0)))
    w2p = jnp.pad(W2, ((0, 0), (0, D2 - W2.shape[1])))
    b1r = b1.reshape(1, 2 * DH)
    b2r = jnp.pad(b2, (0, D2 - b2.shape[0])).reshape(1, D2)

    degp = _degree(dst)

    nblk = NPAD // RB

    g1 = pl.pallas_call(
        _tc_first,
        grid=(nblk,),
        in_specs=[_row_spec(2 * DH), _full_spec((2 * DH, 2 * DH)), _part_spec(DD)],
        out_specs=_part_spec(DH),
        out_shape=jax.ShapeDtypeStruct((NC, NPAD, DH), jnp.float32),
    )(xp, W1, degp)

    p1 = _prop_l1(g1, src1, dst1)

    g2 = pl.pallas_call(
        _tc_mid,
        grid=(nblk,),
        in_specs=[_part_spec(DH), _part_spec(DH),
                  _part_spec(DD), _full_spec((1, 2 * DH)), _full_spec((2 * DH, D2))],
        out_specs=_row_spec(D2),
        out_shape=jax.ShapeDtypeStruct((NPAD, D2), jnp.float32),
    )(p1, g1, degp, b1r, w2p)

    p2 = _prop_l2(g2, src, dst)

    outp = pl.pallas_call(
        _tc_last,
        grid=(nblk,),
        in_specs=[_part_spec(D2), _row_spec(D2), _part_spec(DD),
                  _full_spec((1, D2))],
        out_specs=_row_spec(D2),
        out_shape=jax.ShapeDtypeStruct((NPAD, D2), jnp.float32),
    )(p2, g2, degp, b2r)

    return outp[:N_NODES, :40]


# speedup vs baseline: 2.4426x; 1.0387x over previous
import functools

import jax
import jax.numpy as jnp
from jax import lax
from jax.experimental import pallas as pl
from jax.experimental.pallas import tpu as pltpu
from jax.experimental.pallas import tpu_sc as plsc

N_NODES = 10000
NPAD = 10240
E = 320000
NC, NS = 2, 16
NW = NC * NS
C, B = 80, 128
EPAD = NW * C * B
RPT = NPAD // NS
ZR = 128
DH = 64
D2 = 48
DD = 16

_MESH = plsc.VectorSubcoreMesh(core_axis_name="core", subcore_axis_name="subcore")


def _pipelined_edges(g_p, src_v, dst_v, rows_v, acc, gsems, ssems, n_chunks):
    pltpu.async_copy(g_p.at[src_v.at[0]], rows_v.at[0], gsems[0])
    pltpu.async_copy(g_p.at[src_v.at[1]], rows_v.at[1], gsems[1])

    @pl.loop(0, n_chunks, step=4)
    def _(j):
        for slot in range(4):
            jj = j + slot
            ns = (slot + 2) % 4
            pltpu.make_async_copy(
                g_p.at[src_v.at[jj]], rows_v.at[slot], gsems[slot]).wait()
            pltpu.async_copy(
                rows_v.at[slot], acc.at[dst_v.at[jj]], ssems[slot], add=True)

            @pl.when(jj + 2 < n_chunks)
            def _():
                @pl.when(jj >= 2)
                def _():
                    pltpu.make_async_copy(
                        rows_v.at[ns], acc.at[dst_v.at[jj - 2]],
                        ssems[ns]).wait()
                pltpu.async_copy(
                    g_p.at[src_v.at[jj + 2]], rows_v.at[ns], gsems[ns])

    for k in range(4):
        jj = n_chunks - 4 + k
        pltpu.make_async_copy(
            rows_v.at[jj % 4], acc.at[dst_v.at[jj]], ssems[jj % 4]).wait()


def _make_propagate(D, n_phases):
    out_t = [jax.ShapeDtypeStruct((NC, NPAD, D), jnp.float32)] * n_phases

    @functools.partial(
        pl.kernel,
        out_type=out_t if n_phases > 1 else out_t[0],
        mesh=_MESH,
        compiler_params=pltpu.CompilerParams(use_tc_tiling_on_sc=False),
        scratch_types=[
            pltpu.VMEM((C, B), jnp.int32),
            pltpu.VMEM((C, B), jnp.int32),
            pltpu.VMEM((4, B, D), jnp.float32),
            pltpu.VMEM((ZR, D), jnp.float32),
            pltpu.VMEM_SHARED((NPAD, D), jnp.float32),
        ] + [pltpu.SemaphoreType.DMA] * 8,
    )
    def prop(*refs):
        g_hbm = refs[:n_phases]
        src_hbm, dst_hbm = refs[n_phases], refs[n_phases + 1]
        out_hbm = refs[n_phases + 2:2 * n_phases + 2]
        (src_v, dst_v, rows_v, zbuf, acc, *sems) = refs[2 * n_phases + 2:]
        gsems, ssems = sems[:4], sems[4:]

        cid = lax.axis_index("core")
        sid = lax.axis_index("subcore")
        wid = sid * NC + cid

        @pl.loop(0, ZR)
        def _(r):
            @pl.loop(0, D, step=16)
            def _(k):
                zbuf[r, pl.ds(k, 16)] = jnp.zeros((16,), jnp.float32)

        pltpu.sync_copy(src_hbm.at[wid], src_v)
        pltpu.sync_copy(dst_hbm.at[wid], dst_v)

        for phase in range(n_phases):
            g_p = g_hbm[phase]
            out_p = out_hbm[phase]

            @pl.loop(0, RPT, step=ZR)
            def _(r0):
                pltpu.sync_copy(zbuf, acc.at[pl.ds(sid * RPT + r0, ZR)])

            plsc.subcore_barrier()

            _pipelined_edges(g_p, src_v, dst_v, rows_v, acc, gsems, ssems, C)

            plsc.subcore_barrier()
            pltpu.sync_copy(
                acc.at[pl.ds(sid * RPT, RPT)],
                out_p.at[cid, pl.ds(sid * RPT, RPT)],
            )

    return prop


_prop_l2 = _make_propagate(D2, 1)

C1 = 160
E1PAD = NS * C1 * B


@functools.partial(
    pl.kernel,
    out_type=jax.ShapeDtypeStruct((NC, NPAD, DH), jnp.float32),
    mesh=_MESH,
    compiler_params=pltpu.CompilerParams(use_tc_tiling_on_sc=False),
    scratch_types=[
        pltpu.VMEM((C1, B), jnp.int32),
        pltpu.VMEM((C1, B), jnp.int32),
        pltpu.VMEM((4, B, DH), jnp.float32),
        pltpu.VMEM((ZR, DH), jnp.float32),
        pltpu.VMEM_SHARED((NPAD, DH), jnp.float32),
    ] + [pltpu.SemaphoreType.DMA] * 8,
)
def _prop_l1(g_hbm, src_hbm, dst_hbm, out_hbm, src_v, dst_v, rows_v, zbuf,
             acc, *sems):
    cid = lax.axis_index("core")
    sid = lax.axis_index("subcore")

    @pl.loop(0, ZR)
    def _(r):
        @pl.loop(0, DH, step=16)
        def _(k):
            zbuf[r, pl.ds(k, 16)] = jnp.zeros((16,), jnp.float32)

    pltpu.sync_copy(src_hbm.at[sid], src_v)
    pltpu.sync_copy(dst_hbm.at[sid], dst_v)
    g_p = g_hbm.at[cid]

    @pl.loop(0, RPT, step=ZR)
    def _(r0):
        pltpu.sync_copy(zbuf, acc.at[pl.ds(sid * RPT + r0, ZR)])

    gsems, ssems = sems[:4], sems[4:]
    plsc.subcore_barrier()

    _pipelined_edges(g_p, src_v, dst_v, rows_v, acc, gsems, ssems, C1)

    plsc.subcore_barrier()
    pltpu.sync_copy(
        acc.at[pl.ds(sid * RPT, RPT)],
        out_hbm.at[cid, pl.ds(sid * RPT, RPT)],
    )


@functools.partial(
    pl.kernel,
    out_type=jax.ShapeDtypeStruct((NC, NPAD, DD), jnp.float32),
    mesh=_MESH,
    compiler_params=pltpu.CompilerParams(use_tc_tiling_on_sc=False),
    scratch_types=[
        pltpu.VMEM((C, B), jnp.int32),
        pltpu.VMEM((B, DD), jnp.float32),
        pltpu.VMEM((ZR, DD), jnp.float32),
        pltpu.VMEM_SHARED((NPAD, DD), jnp.float32),
    ],
)
def _degree(dst_hbm, out_hbm, dst_v, ones_v, zbuf, acc):
    cid = lax.axis_index("core")
    sid = lax.axis_index("subcore")
    wid = sid * NC + cid

    @pl.loop(0, ZR)
    def _(r):
        zbuf[r, pl.ds(0, 16)] = jnp.zeros((16,), jnp.float32)

    @pl.loop(0, B)
    def _(r):
        ones_v[r, pl.ds(0, 16)] = jnp.full((16,), 1.0 / 16.0, jnp.float32)

    @pl.loop(0, RPT, step=ZR)
    def _(r0):
        pltpu.sync_copy(zbuf, acc.at[pl.ds(sid * RPT + r0, ZR)])

    pltpu.sync_copy(dst_hbm.at[wid], dst_v)
    plsc.subcore_barrier()

    @pl.loop(0, C)
    def _(j):
        pltpu.sync_copy(ones_v, acc.at[dst_v.at[j]], add=True)

    plsc.subcore_barrier()
    pltpu.sync_copy(
        acc.at[pl.ds(sid * RPT, RPT)],
        out_hbm.at[cid, pl.ds(sid * RPT, RPT)],
    )


RB = 2048


def _dis_from_parts(d):
    deg = jnp.sum(d[0], axis=1) + jnp.sum(d[1], axis=1) + 1.0
    return lax.rsqrt(deg)


def _tc_first(x_ref, w_ref, d_ref, o_ref):
    dis = _dis_from_parts(d_ref[...])
    y = jnp.dot(x_ref[...], w_ref[...], preferred_element_type=jnp.float32,
                precision=lax.Precision.HIGHEST)
    g = y * dis[:, None]
    o_ref[0] = g[:, :DH]
    o_ref[1] = g[:, DH:]


def _tc_mid(p_ref, g_ref, d_ref, b_ref, w_ref, o_ref):
    dis = _dis_from_parts(d_ref[...])[:, None]
    sa = p_ref[0] + g_ref[0]
    sb = p_ref[1] + g_ref[1]
    s = jnp.concatenate([sa, sb], axis=1)
    h = jnp.maximum(dis * s + b_ref[...], 0.0)
    y = jnp.dot(h, w_ref[...], preferred_element_type=jnp.float32,
                precision=lax.Precision.HIGHEST)
    o_ref[...] = y * dis


def _tc_last(p_ref, g_ref, d_ref, b_ref, o_ref):
    dis = _dis_from_parts(d_ref[...])[:, None]
    s = p_ref[0] + p_ref[1] + g_ref[...]
    o_ref[...] = dis * s + b_ref[...]


def _row_spec(d):
    return pl.BlockSpec((RB, d), lambda i: (i, 0))


def _part_spec(d):
    return pl.BlockSpec((NC, RB, d), lambda i: (0, i, 0))


def _full_spec(shape):
    return pl.BlockSpec(shape, lambda i: tuple(0 for _ in shape))


@jax.jit
def kernel(x, edge_index, W1, b1, W2, b2):
    ei = edge_index.astype(jnp.int32)
    pad_src = N_NODES + jnp.arange(EPAD - E, dtype=jnp.int32) % (NPAD - N_NODES)
    pad_dst = N_NODES + jnp.arange(EPAD - E, dtype=jnp.int32) % (NPAD - N_NODES)
    src = jnp.concatenate([ei[0], pad_src]).reshape(NW, C, B)
    dst = jnp.concatenate([ei[1], pad_dst]).reshape(NW, C, B)
    pad1_src = N_NODES + jnp.arange(E1PAD - E, dtype=jnp.int32) % (NPAD - N_NODES)
    pad1_dst = N_NODES + jnp.arange(E1PAD - E, dtype=jnp.int32) % (NPAD - N_NODES)
    src1 = jnp.concatenate([ei[0], pad1_src]).reshape(NS, C1, B)
    dst1 = jnp.concatenate([ei[1], pad1_dst]).reshape(NS, C1, B)
    xp = jnp.pad(x, ((0, NPAD - N_NODES), (0, 0)))
    w2p = jnp.pad(W2, ((0, 0), (0, D2 - W2.shape[1])))
    b1r = b1.reshape(1, 2 * DH)
    b2r = jnp.pad(b2, (0, D2 - b2.shape[0])).reshape(1, D2)

    degp = _degree(dst)

    nblk = NPAD // RB

    g1 = pl.pallas_call(
        _tc_first,
        grid=(nblk,),
        in_specs=[_row_spec(2 * DH), _full_spec((2 * DH, 2 * DH)), _part_spec(DD)],
        out_specs=_part_spec(DH),
        out_shape=jax.ShapeDtypeStruct((NC, NPAD, DH), jnp.float32),
    )(xp, W1, degp)

    p1 = _prop_l1(g1, src1, dst1)

    g2 = pl.pallas_call(
        _tc_mid,
        grid=(nblk,),
        in_specs=[_part_spec(DH), _part_spec(DH),
                  _part_spec(DD), _full_spec((1, 2 * DH)), _full_spec((2 * DH, D2))],
        out_specs=_row_spec(D2),
        out_shape=jax.ShapeDtypeStruct((NPAD, D2), jnp.float32),
    )(p1, g1, degp, b1r, w2p)

    p2 = _prop_l2(g2, src, dst)

    outp = pl.pallas_call(
        _tc_last,
        grid=(nblk,),
        in_specs=[_part_spec(D2), _row_spec(D2), _part_spec(DD),
                  _full_spec((1, D2))],
        out_specs=_row_spec(D2),
        out_shape=jax.ShapeDtypeStruct((NPAD, D2), jnp.float32),
    )(p2, g2, degp, b2r)

    return outp[:N_NODES, :40]
